# baseline (speedup 1.0000x reference)
import functools

import jax
import jax.numpy as jnp
from jax import lax
from jax.experimental import pallas as pl
from jax.experimental.pallas import tpu as pltpu
from jax.experimental.pallas import tpu_sc as plsc

_B, _L, _D, _E, _R = 8, 512, 512, 16384, 64
_LN1_EPS = 1e-06
_LN2_EPS = 1e-05
_INV_SQRT_D = 1.0 / (512.0 ** 0.5)

_NT = 8
_TR = _L // _NT
_NW = 32

_CT = (((1,), (1,)), ((), ()))


def _make_pre_body(hh):
    def _pre_body(h_ref, ei_ref, r_ref, ln1w_ref, ln1b_ref, wq_ref,
                  bq_ref, wk_ref, wv_ref, rel_ref,
                  xn_ref, v_ref, s_ref, qr_ref, pk_ref, *seed_ref):
        _pre_common(hh, h_ref, ei_ref, r_ref, ln1w_ref, ln1b_ref, wq_ref,
                    bq_ref, wk_ref, wv_ref, rel_ref,
                    xn_ref, v_ref, s_ref, qr_ref, pk_ref,
                    seed_ref[0] if seed_ref else None)
    return _pre_body


def _pre_common(hh, h_ref, ei_ref, r_ref, ln1w_ref, ln1b_ref, wq_ref,
                bq_ref, wk_ref, wv_ref, rel_ref,
                xn_ref, v_ref, s_ref, qr_ref, pk_ref, seed_ref):
    x = h_ref[0]
    mu = jnp.mean(x, axis=-1, keepdims=True)
    var = jnp.mean((x - mu) ** 2, axis=-1, keepdims=True)
    xn = (x - mu) / jnp.sqrt(var + _LN1_EPS) * ln1w_ref[...] + ln1b_ref[...]
    q = lax.dot_general(xn, wq_ref[...], _CT,
                        preferred_element_type=jnp.float32) + bq_ref[...]
    k = lax.dot_general(xn, wk_ref[...], _CT, preferred_element_type=jnp.float32)
    v = lax.dot_general(xn, wv_ref[...], _CT, preferred_element_type=jnp.float32)
    s = lax.dot_general(q, k, _CT,
                        preferred_element_type=jnp.float32) * _INV_SQRT_D
    qr = lax.dot_general(q, rel_ref[...], _CT,
                         preferred_element_type=jnp.float32) * _INV_SQRT_D
    c = jnp.max(s, axis=1, keepdims=True) + jnp.max(qr, axis=1, keepdims=True)
    xn_ref[0] = xn
    v_ref[0] = v
    if seed_ref is not None:
        seed_ref[0] = xn
    s_ref[0] = jnp.exp(s - c)
    qr_ref[0] = jnp.exp(qr)

    @pl.when(pl.program_id(0) == 0)
    def _():
        rv = r_ref[pl.ds(hh * (_B // 2), _B // 2), :]
        pk_ref[...] = (ei_ref[:, 1] << 15) | (ei_ref[:, 0] << 6) | rv


def _post_body(a_ref, p_ref, v_ref, rel_ref, wo_ref, bo_ref, xn_ref,
               ln2w_ref, ln2b_ref, h_ref, prev_ref, out_ref):
    del prev_ref
    amat = a_ref[0]
    denom = jnp.sum(amat, axis=1, keepdims=True)
    agg = (jnp.dot(amat.astype(jnp.bfloat16), v_ref[0].astype(jnp.bfloat16),
                   preferred_element_type=jnp.float32)
           + jnp.dot(p_ref[0].astype(jnp.bfloat16), rel_ref[...],
                     preferred_element_type=jnp.float32))
    agg = agg / jnp.maximum(denom, 1e-12)
    o = lax.dot_general(agg.astype(jnp.bfloat16), wo_ref[...], _CT,
                        preferred_element_type=jnp.float32) + bo_ref[...]
    y = xn_ref[0] + o
    mu = jnp.mean(y, axis=-1, keepdims=True)
    var = jnp.mean((y - mu) ** 2, axis=-1, keepdims=True)
    z = (y - mu) / jnp.sqrt(var + _LN2_EPS) * ln2w_ref[...] + ln2b_ref[...]
    out_ref[0] = h_ref[0] + jnp.where(z > 0, z, jnp.exp(jnp.minimum(z, 0.0)) - 1.0)


def _edge_kernel_body(nb, s_hbm, qr_hbm, pk_hbm, a_hbm, p_hbm,
                      s_v, qr_v, a_v, p_v, pk_v, sem):
    wid = lax.axis_index("c") * 16 + lax.axis_index("s")

    def assignment(j, carry):
        a = wid + _NW * j
        b = a // _NT
        t = a % _NT
        lo = t * _TR
        c1 = pltpu.async_copy(s_hbm.at[b, pl.ds(lo, _TR)], s_v, sem)
        c2 = pltpu.async_copy(qr_hbm.at[b, pl.ds(lo, _TR)], qr_v, sem)
        c5 = pltpu.async_copy(pk_hbm.at[b], pk_v, sem)

        z16 = jnp.zeros((16,), jnp.float32)

        def zero_all(i):
            rr = i >> 5
            cc = (i & 31) * 16
            a_v[rr, pl.ds(cc, 16)] = z16

        def zero_p(i):
            rr = i >> 2
            cc = (i & 3) * 16
            p_v[rr, pl.ds(cc, 16)] = z16

        plsc.parallel_loop(0, (_TR * _L) // 16, 1, unroll=8)(zero_all)
        plsc.parallel_loop(0, (_TR * _R) // 16, 1, unroll=8)(zero_p)
        c1.wait(); c2.wait(); c5.wait()

        def vec(i):
            p = pk_v[pl.ds(i * 16, 16)]
            d = p >> 15
            dloc = d - lo
            m = (dloc >= 0) & (dloc < _TR)
            dl = dloc & (_TR - 1)
            src = (p >> 6) & 511
            r = p & 63
            sv = plsc.load_gather(s_v, [dl, src], mask=m)
            qv = plsc.load_gather(qr_v, [dl, r], mask=m)
            ex = sv * qv
            plsc.addupdate_scatter(a_v, [dl, src], ex, mask=m)
            plsc.addupdate_scatter(p_v, [dl, r], ex, mask=m)

        plsc.parallel_loop(0, _E // 16, 1, unroll=8)(vec)

        pltpu.sync_copy(a_v, a_hbm.at[b, pl.ds(lo, _TR)])
        pltpu.sync_copy(p_v, p_hbm.at[b, pl.ds(lo, _TR)])
        return carry

    lax.fori_loop(0, (nb * _NT) // _NW, assignment, 0)


@functools.lru_cache(maxsize=2)
def _edge_kernel(nb):
    mesh = plsc.VectorSubcoreMesh(core_axis_name="c", subcore_axis_name="s")
    return pl.kernel(
        functools.partial(_edge_kernel_body, nb),
        out_type=(jax.ShapeDtypeStruct((nb, _L, _L), jnp.float32),
                  jax.ShapeDtypeStruct((nb, _L, _R), jnp.float32)),
        mesh=mesh,
        scratch_types=[
            pltpu.VMEM((_TR, _L), jnp.float32),
            pltpu.VMEM((_TR, _R), jnp.float32),
            pltpu.VMEM((_TR, _L), jnp.float32),
            pltpu.VMEM((_TR, _R), jnp.float32),
            pltpu.VMEM((_E,), jnp.int32),
            pltpu.SemaphoreType.DMA,
        ],
        compiler_params=pltpu.CompilerParams(needs_layout_passes=False),
    )


def kernel(hidden_states, edge_index, edge_rel, rel_table, Wq, bq, Wk, Wv,
           Wo, bo, ln1_w, ln1_b, ln2_w, ln2_b):
    ei = edge_index.astype(jnp.int32)
    rel = edge_rel.astype(jnp.int32)
    bf = lambda a: a.astype(jnp.bfloat16)

    row = lambda a: a.reshape(1, -1)
    full2 = lambda shape: pl.BlockSpec(shape, lambda b: (0, 0))
    per_g = lambda shape: pl.BlockSpec(shape, lambda b: (b, 0, 0))

    _HB = _B // 2
    out = None
    for hh in (0, 1):
        off_g = lambda shape, _h=hh: pl.BlockSpec(
            shape, lambda b, __h=_h: (b + _HB * __h, 0, 0))
        outs = pl.pallas_call(
            _make_pre_body(hh),
            grid=(_HB,),
            in_specs=[
                off_g((1, _L, _D)),
                pl.BlockSpec((_HB, 2, _E), lambda b, _h=hh: (_h, 0, 0)),
                full2((_B, _E)),
                full2((1, _D)), full2((1, _D)),
                full2((_D, _D)), full2((1, _D)),
                full2((_D, _D)), full2((_D, _D)),
                full2((_R, _D)),
            ],
            out_specs=[
                per_g((1, _L, _D)), per_g((1, _L, _D)),
                per_g((1, _L, _L)), per_g((1, _L, _R)),
                pl.BlockSpec((_HB, _E), lambda b: (0, 0)),
            ] + ([per_g((1, _L, _D))] if hh == 0 else []),
            out_shape=[
                jax.ShapeDtypeStruct((_HB, _L, _D), jnp.float32),
                jax.ShapeDtypeStruct((_HB, _L, _D), jnp.float32),
                jax.ShapeDtypeStruct((_HB, _L, _L), jnp.float32),
                jax.ShapeDtypeStruct((_HB, _L, _R), jnp.float32),
                jax.ShapeDtypeStruct((_HB, _E), jnp.int32),
            ] + ([jax.ShapeDtypeStruct((_B, _L, _D), jnp.float32)]
                 if hh == 0 else []),
        )(hidden_states, ei, rel, row(ln1_w), row(ln1_b), Wq, row(bq),
          Wk, Wv, rel_table)

        xn, v, s, qr, pk = outs[:5]
        if out is None:
            out = outs[5]

        a_mat, p_mat = _edge_kernel(_HB)(s, qr, pk)

        out = pl.pallas_call(
            _post_body,
            grid=(_HB,),
            in_specs=[
                per_g((1, _L, _L)), per_g((1, _L, _R)), per_g((1, _L, _D)),
                full2((_R, _D)), full2((_D, _D)), full2((1, _D)),
                per_g((1, _L, _D)),
                full2((1, _D)), full2((1, _D)),
                off_g((1, _L, _D)),
                pl.BlockSpec(memory_space=pl.ANY),
            ],
            out_specs=off_g((1, _L, _D)),
            out_shape=jax.ShapeDtypeStruct((_B, _L, _D), jnp.float32),
            input_output_aliases={10: 0},
        )(a_mat, p_mat, v, bf(rel_table), bf(Wo), row(bo), xn,
          row(ln2_w), row(ln2_b), hidden_states, out)

    return out
